# Initial kernel scaffold; baseline (speedup 1.0000x reference)
import jax
import jax.numpy as jnp
from jax.experimental import pallas as pl


def kernel(x, edge_index, W, b):
    raise NotImplementedError("write your pallas kernel here")


# speedup vs baseline: 4.9063x; 4.9063x over previous
import functools

import jax
import jax.numpy as jnp
from jax import lax
from jax.experimental import pallas as pl
from jax.experimental.pallas import tpu as pltpu
from jax.experimental.pallas import tpu_sc as plsc

_NC = 2
_NS = 16
_NW = _NC * _NS
_L = 16

_CHUNK = 128


def _sc_segment_sum(x_pad, src, dst, n_pad):
    e = src.shape[0]
    d = x_pad.shape[1]
    ew = e // _NW
    chunk = _CHUNK
    nchunks = ew // chunk
    rpt = n_pad // _NS

    mesh = plsc.VectorSubcoreMesh(core_axis_name="c", subcore_axis_name="s")

    @functools.partial(
        pl.kernel,
        out_type=(
            jax.ShapeDtypeStruct((_NC, n_pad, d), jnp.float32),
            jax.ShapeDtypeStruct((_NW, n_pad), jnp.float32),
        ),
        mesh=mesh,
        scratch_types=(
            pltpu.VMEM((chunk,), jnp.int32),
            pltpu.VMEM((chunk,), jnp.int32),
            pltpu.VMEM((chunk,), jnp.int32),
            pltpu.VMEM((chunk, d), jnp.float32),
            pltpu.VMEM((n_pad,), jnp.float32),
            pltpu.VMEM((chunk,), jnp.float32),
            pltpu.VMEM_SHARED((n_pad, d), jnp.float32),
            pltpu.VMEM_SHARED((_NS * n_pad,), jnp.float32),
            pltpu.SemaphoreType.DMA,
        ),
    )
    def run(x_hbm, src_hbm, dst_hbm, acc_hbm, deg_hbm,
            srcbuf, dstbuf, dstbuf2, rows, degbuf, onesbuf, acc_sh, deg_sh,
            sem):
        c = lax.axis_index("c")
        s = lax.axis_index("s")
        w = s * _NC + c

        zero16 = jnp.zeros((_L,), jnp.float32)
        one16 = jnp.ones((_L,), jnp.float32)

        @pl.loop(0, chunk)
        def _(i):
            for j in range(d // _L):
                rows[i, pl.ds(j * _L, _L)] = zero16

        @pl.loop(0, n_pad // _L)
        def _(i):
            degbuf[pl.ds(i * _L, _L)] = zero16

        @pl.loop(0, chunk // _L)
        def _(i):
            onesbuf[pl.ds(i * _L, _L)] = one16

        base = s * rpt
        off = 0
        while off < rpt:
            step = min(chunk, rpt - off)
            pltpu.sync_copy(rows.at[pl.ds(0, step)],
                            acc_sh.at[pl.ds(base + off, step)])
            off += step
        pltpu.sync_copy(degbuf, deg_sh.at[pl.ds(s * n_pad, n_pad)])
        plsc.subcore_barrier()

        ebase = w * ew

        @pl.loop(0, nchunks)
        def _(ci):
            eoff = ebase + ci * chunk
            pltpu.sync_copy(src_hbm.at[pl.ds(eoff, chunk)], srcbuf)
            pltpu.sync_copy(dst_hbm.at[pl.ds(eoff, chunk)], dstbuf)
            pltpu.async_copy(x_hbm.at[srcbuf], rows, sem).wait()
            pltpu.sync_copy(rows, acc_sh.at[dstbuf], add=True)
            for j in range(chunk // _L):
                dstbuf2[pl.ds(j * _L, _L)] = (
                    dstbuf[pl.ds(j * _L, _L)] + s * n_pad)
            pltpu.sync_copy(onesbuf, deg_sh.at[dstbuf2], add=True)

        plsc.subcore_barrier()
        pltpu.sync_copy(acc_sh.at[pl.ds(base, rpt)],
                        acc_hbm.at[c, pl.ds(base, rpt)])
        pltpu.sync_copy(deg_sh.at[pl.ds(s * n_pad, n_pad)], deg_hbm.at[w])

    return run(x_pad, src, dst)


def _tc_combine(x_pad, W, b2, acc, deg):
    n_pad, d = x_pad.shape
    o = W.shape[1]
    bm = 2048 if n_pad % 2048 == 0 else 128

    def body(x_ref, w_ref, b_ref, acc_ref, deg_ref, o_ref):
        xb = x_ref[...]
        a = acc_ref[0] + acc_ref[1]
        dg = jnp.sum(deg_ref[...], axis=0)
        neigh = a / jnp.clip(dg, 1.0, None)[:, None]
        w1 = w_ref[pl.ds(0, d), :]
        w2 = w_ref[pl.ds(d, d), :]
        o_ref[...] = (
            jnp.dot(xb, w1, preferred_element_type=jnp.float32)
            + jnp.dot(neigh, w2, preferred_element_type=jnp.float32)
            + b_ref[...]
        )

    return pl.pallas_call(
        body,
        grid=(n_pad // bm,),
        in_specs=[
            pl.BlockSpec((bm, d), lambda i: (i, 0)),
            pl.BlockSpec((2 * d, o), lambda i: (0, 0)),
            pl.BlockSpec((1, o), lambda i: (0, 0)),
            pl.BlockSpec((_NC, bm, d), lambda i: (0, i, 0)),
            pl.BlockSpec((_NW, bm), lambda i: (0, i)),
        ],
        out_specs=pl.BlockSpec((bm, o), lambda i: (i, 0)),
        out_shape=jax.ShapeDtypeStruct((n_pad, o), jnp.float32),
    )(x_pad, W, b2, acc, deg)


def kernel(x, edge_index, W, b):
    n, d = x.shape
    granule = _NS * 128
    n_pad = ((n + granule - 1) // granule) * granule
    if n_pad == n:
        n_pad += granule
    x_pad = jnp.pad(x, ((0, n_pad - n), (0, 0)))
    src = edge_index[0].astype(jnp.int32)
    dst = edge_index[1].astype(jnp.int32)
    e = src.shape[0]
    e_p = -(-e // (_NW * _CHUNK)) * (_NW * _CHUNK)
    if e_p != e:
        src = jnp.pad(src, (0, e_p - e))
        dst = jnp.pad(dst, (0, e_p - e), constant_values=n)
    acc, deg = _sc_segment_sum(x_pad, src, dst, n_pad)
    out = _tc_combine(x_pad, W, b.reshape(1, -1), acc, deg)
    return out[:n]
